# baseline (speedup 1.0000x reference)
import functools

import jax
import jax.numpy as jnp
from jax import lax
from jax.experimental import pallas as pl
from jax.experimental.pallas import tpu as pltpu
from jax.experimental.pallas import tpu_sc as plsc

D = 32
B = 4096
H = 200

NC, NS = 2, 16
NW = NC * NS
BPW = B // NW
NPAIR = BPW // 2

_mesh = plsc.VectorSubcoreMesh(core_axis_name="c", subcore_axis_name="s")


@functools.partial(
    pl.kernel,
    out_type=jax.ShapeDtypeStruct((B // 4, 128), jnp.float32),
    mesh=_mesh,
    scratch_types=[
        pltpu.VMEM((BPW * H,), jnp.int32),
        pltpu.VMEM((BPW * H,), jnp.int32),
        pltpu.VMEM((H, 128), jnp.float32),
        pltpu.VMEM((H, 128), jnp.float32),
        pltpu.VMEM((BPW // 4, 128), jnp.float32),
        pltpu.SemaphoreType.DMA,
        pltpu.SemaphoreType.DMA,
    ],
)
def _cbow_sc(x_hbm, w_hbm, out_hbm, idx_v, slot_v, buf0, buf1, out_v,
             sem0, sem1):
    wid = lax.axis_index("s") * NC + lax.axis_index("c")
    base = wid * BPW
    pltpu.sync_copy(x_hbm.at[pl.ds(base * H, BPW * H)], idx_v)

    def conv_body(k, _):
        v = idx_v[pl.ds(k * 16, 16)]
        slot_v[pl.ds(k * 16, 16)] = v >> 2
        return 0

    lax.fori_loop(0, BPW * H // 16, conv_body, 0, unroll=8)

    bufs = (buf0, buf1)
    sems = (sem0, sem1)

    def gather(g, b):
        return pltpu.async_copy(
            w_hbm.at[slot_v.at[pl.ds(g * H, H)]], bufs[b], sems[b])

    gather(0, 0)
    gather(1, 1)

    def pair_body(j, _):
        for b in range(2):
            g = j * 2 + b
            row0 = g * H
            pltpu.make_async_copy(
                w_hbm.at[slot_v.at[pl.ds(0, H)]], bufs[b], sems[b]).wait()
            buf = bufs[b]

            def blk_body(kb, carry, buf=buf, row0=row0):
                a0, a1 = carry
                qv = (idx_v[pl.ds(row0 + kb * 16, 16)] & 3) * 32
                r0 = kb * 16
                for jj in range(16):
                    qo = qv[jj]
                    a0 = a0 + buf[r0 + jj, pl.ds(qo, 16)]
                    a1 = a1 + buf[r0 + jj, pl.ds(qo + 16, 16)]
                return a0, a1

            zero = jnp.zeros((16,), jnp.float32)
            a0, a1 = lax.fori_loop(0, 12, blk_body, (zero, zero))
            qv = (idx_v[pl.ds(row0 + 184, 16)] & 3) * 32
            for jj in range(8):
                qo = qv[8 + jj]
                a0 = a0 + buf[192 + jj, pl.ds(qo, 16)]
                a1 = a1 + buf[192 + jj, pl.ds(qo + 16, 16)]
            out_v[g // 4, pl.ds((g % 4) * 32, 16)] = a0
            out_v[g // 4, pl.ds((g % 4) * 32 + 16, 16)] = a1

            @pl.when(g + 2 < BPW)
            def _():
                gather(g + 2, b)
        return 0

    lax.fori_loop(0, NPAIR, pair_body, 0)
    pltpu.sync_copy(out_v, out_hbm.at[pl.ds(wid * (BPW // 4), BPW // 4)])


def kernel(x, W):
    flat_x = x.reshape(-1).astype(jnp.int32)
    w128 = W.reshape(250000, 128)
    return _cbow_sc(flat_x, w128).reshape(B, D)


# speedup vs baseline: 1.8680x; 1.8680x over previous
import functools

import jax
import jax.numpy as jnp
from jax import lax
from jax.experimental import pallas as pl
from jax.experimental.pallas import tpu as pltpu
from jax.experimental.pallas import tpu_sc as plsc

NUM_TOKENS = 1000000
D = 32
B = 4096
H = 200

NC, NS = 2, 16
NW = NC * NS
BPW = B // NW
G = 4
ROWS_G = G * H
NGROUPS = BPW // G

_mesh = plsc.VectorSubcoreMesh(core_axis_name="c", subcore_axis_name="s")


@functools.partial(
    pl.kernel,
    out_type=jax.ShapeDtypeStruct((B, D), jnp.float32),
    mesh=_mesh,
    scratch_types=[
        pltpu.VMEM((BPW * H,), jnp.int32),
        pltpu.VMEM((ROWS_G, D), jnp.float32),
        pltpu.VMEM((ROWS_G, D), jnp.float32),
        pltpu.VMEM((BPW, D), jnp.float32),
        pltpu.SemaphoreType.DMA,
        pltpu.SemaphoreType.DMA,
    ],
    compiler_params=pltpu.CompilerParams(use_tc_tiling_on_sc=False),
)
def _cbow_sc(x_hbm, w_hbm, out_hbm, idx_v, buf0, buf1, out_v, sem0, sem1):
    wid = lax.axis_index("s") * NC + lax.axis_index("c")
    base = wid * BPW
    pltpu.sync_copy(x_hbm.at[pl.ds(base * H, BPW * H)], idx_v)

    bufs = (buf0, buf1)
    sems = (sem0, sem1)
    copies = [None, None]
    copies[0] = pltpu.async_copy(
        w_hbm.at[idx_v.at[pl.ds(0, ROWS_G)]], bufs[0], sems[0])
    for g in range(NGROUPS):
        cur = g % 2
        copies[cur].wait()
        if g + 1 < NGROUPS:
            nxt = (g + 1) % 2
            copies[nxt] = pltpu.async_copy(
                w_hbm.at[idx_v.at[pl.ds((g + 1) * ROWS_G, ROWS_G)]],
                bufs[nxt], sems[nxt])
        buf = bufs[cur]
        for i in range(G):
            row0 = i * H

            def h_body(h, carry, buf=buf, row0=row0):
                a0, a1 = carry
                a0 = a0 + buf[row0 + h, pl.ds(0, 16)]
                a1 = a1 + buf[row0 + h, pl.ds(16, 16)]
                return a0, a1

            zero = jnp.zeros((16,), jnp.float32)
            a0, a1 = lax.fori_loop(0, H, h_body, (zero, zero), unroll=8)
            out_v[g * G + i, pl.ds(0, 16)] = a0
            out_v[g * G + i, pl.ds(16, 16)] = a1

    pltpu.sync_copy(out_v, out_hbm.at[pl.ds(base, BPW)])


def kernel(x, W):
    from jax.experimental.layout import Format, Layout, with_layout_constraint

    flat_x = x.reshape(-1).astype(jnp.int32)
    del Format
    wl = with_layout_constraint(
        W, Layout(major_to_minor=(0, 1), tiling=((8,),)))
    return _cbow_sc(flat_x, wl)
